# baseline (speedup 1.0000x reference)
import functools

import jax
import jax.numpy as jnp
from jax import lax
from jax.experimental import pallas as pl
from jax.experimental.pallas import tpu as pltpu
from jax.experimental.pallas import tpu_sc as plsc

EPS_ = 1e-07
MAX_NORM_ = 1 - 0.0001

NUM_WORKERS = 32
IDX_ROWS_PER_CHUNK = 8
D = 16
LANE = 128


def _project_chunk(gbuf, sbuf, n_rows, seq_len):
    lane = lax.iota(jnp.int32, 16)

    def block(b, carry):
        f = lane + b * 16
        d0 = f // seq_len
        d1 = f % seq_len
        diags = []
        ssum = jnp.zeros((16,), jnp.float32)
        for j in range(D):
            d2 = (lane + j) & (D - 1)
            dg = plsc.load_gather(gbuf, [d0, d1, d2])
            diags.append(dg)
            ssum = ssum + dg * dg
        bits = lax.bitcast_convert_type(ssum, jnp.int32)
        y = lax.bitcast_convert_type(
            jnp.int32(0x5F3759DF) - (bits >> 1), jnp.float32)
        for _ in range(3):
            y = y * (1.5 - 0.5 * ssum * y * y)
        norm = ssum * y
        factor = jnp.where(norm >= MAX_NORM_, MAX_NORM_ * y,
                           jnp.ones((16,), jnp.float32))
        for j in range(D):
            d2 = (lane + j) & (D - 1)
            plsc.store_scatter(sbuf, [d0 * D + d2, d1], diags[j] * factor)
        return carry

    lax.fori_loop(0, n_rows // 16, block, 0)


def _make_sc_kernel(n_idx, seq_len):
    idx_rows_per_worker = n_idx // NUM_WORKERS
    chunks = idx_rows_per_worker // IDX_ROWS_PER_CHUNK
    rows_per_chunk = IDX_ROWS_PER_CHUNK * seq_len
    out_rows_per_chunk = IDX_ROWS_PER_CHUNK * D
    info = plsc.get_sparse_core_info()
    nc = info.num_cores
    mesh = plsc.VectorSubcoreMesh(core_axis_name="c", subcore_axis_name="s")
    gbuf_t = pltpu.VMEM((IDX_ROWS_PER_CHUNK, seq_len, D), jnp.float32)
    sbuf_t = pltpu.VMEM((out_rows_per_chunk, LANE), jnp.float32)

    @functools.partial(
        pl.kernel,
        mesh=mesh,
        out_type=jax.ShapeDtypeStruct((n_idx * D, LANE), jnp.float32),
        compiler_params=pltpu.CompilerParams(needs_layout_passes=False,
                                             use_tc_tiling_on_sc=False),
        scratch_types=[
            pltpu.VMEM((seq_len, idx_rows_per_worker), jnp.int32),
            pltpu.VMEM((idx_rows_per_worker, seq_len), jnp.int32),
            gbuf_t,
            gbuf_t,
            sbuf_t,
            sbuf_t,
            pltpu.SemaphoreType.DMA,
            pltpu.SemaphoreType.DMA,
            pltpu.SemaphoreType.DMA,
            pltpu.SemaphoreType.DMA,
        ],
    )
    def sc_kernel(idx_hbm, emb_hbm, out_hbm, idx_tv, idx_v, ga, gb, sa, sb,
                  gsem_a, gsem_b, ssem_a, ssem_b):
        wid = lax.axis_index("s") * nc + lax.axis_index("c")
        base = wid * idx_rows_per_worker
        out_row0 = base * D
        pltpu.sync_copy(idx_hbm.at[:, pl.ds(base, idx_rows_per_worker)],
                        idx_tv)
        lane = lax.iota(jnp.int32, 16)
        zeros16 = jnp.zeros((16,), jnp.float32)

        def repack(c, carry):
            col = c * 16 + lane
            for j in range(seq_len):
                jj = jnp.full((16,), j, jnp.int32)
                v = plsc.load_gather(idx_tv, [jj, col])
                plsc.store_scatter(idx_v, [col, jj], v)
            return carry

        lax.fori_loop(0, idx_rows_per_worker // 16, repack, 0)


        def zrow(r, carry):
            for k in range(LANE // 16):
                plsc.store_scatter(sa, [jnp.full((16,), r, jnp.int32),
                                        lane + k * 16], zeros16)
                plsc.store_scatter(sb, [jnp.full((16,), r, jnp.int32),
                                        lane + k * 16], zeros16)
            return carry

        lax.fori_loop(0, out_rows_per_chunk, zrow, 0)

        def start_gathers(c, buf, gsem):
            r0 = c * IDX_ROWS_PER_CHUNK
            for k in range(IDX_ROWS_PER_CHUNK):
                pltpu.make_async_copy(
                    emb_hbm.at[idx_v.at[r0 + k]], buf.at[k], gsem).start()

        def wait_gathers(c, buf, gsem):
            r0 = c * IDX_ROWS_PER_CHUNK
            for k in range(IDX_ROWS_PER_CHUNK):
                pltpu.make_async_copy(
                    emb_hbm.at[idx_v.at[r0 + k]], buf.at[k], gsem).wait()

        def store_of(c, buf, ssem):
            return pltpu.make_async_copy(
                buf,
                out_hbm.at[pl.ds(out_row0 + c * out_rows_per_chunk,
                                 out_rows_per_chunk)],
                ssem)

        start_gathers(0, ga, gsem_a)

        def halfstep(c, gbuf, gsem, sbuf, ssem, ngbuf, ngsem):
            wait_gathers(c, gbuf, gsem)

            @pl.when(c + 1 < chunks)
            def _():
                start_gathers(c + 1, ngbuf, ngsem)

            @pl.when(c >= 2)
            def _():
                store_of(c - 2, sbuf, ssem).wait()

            _project_chunk(gbuf, sbuf, rows_per_chunk, seq_len)
            store_of(c, sbuf, ssem).start()

        def step(t, carry):
            halfstep(2 * t, ga, gsem_a, sa, ssem_a, gb, gsem_b)
            halfstep(2 * t + 1, gb, gsem_b, sb, ssem_b, ga, gsem_a)
            return carry

        lax.fori_loop(0, chunks // 2, step, 0)
        store_of(chunks - 2, sa, ssem_a).wait()
        store_of(chunks - 1, sb, ssem_b).wait()

    return sc_kernel


DETR_BN = 1000


def _detranspose_table(emb):
    v, d = emb.shape
    g8 = 128 // d
    emb_t3 = emb.T.reshape(d, v // g8, g8)
    bn = DETR_BN

    def body(in_ref, out_ref):
        out_ref[...] = in_ref[...].transpose(1, 2, 0).reshape(bn, 128)

    out = pl.pallas_call(
        body,
        grid=(v // g8 // bn,),
        in_specs=[pl.BlockSpec((d, bn, g8), lambda g: (0, g, 0))],
        out_specs=pl.BlockSpec((bn, 128), lambda g: (g, 0)),
        out_shape=jax.ShapeDtypeStruct((v * d // 128, 128), jnp.float32),
    )(emb_t3)
    return out.reshape(v, d)


def kernel(idx, emb):
    n_idx, seq_len = idx.shape
    out2 = _make_sc_kernel(n_idx, seq_len)(idx.astype(jnp.int32).T,
                                           _detranspose_table(emb))
    out3 = out2.reshape(n_idx, D, LANE)[:, :, :seq_len]
    return out3.transpose(0, 2, 1)


# speedup vs baseline: 3.4801x; 3.4801x over previous
import functools

import jax
import jax.numpy as jnp
from jax import lax
from jax.experimental import pallas as pl
from jax.experimental.pallas import tpu as pltpu
from jax.experimental.pallas import tpu_sc as plsc

EPS_ = 1e-07
MAX_NORM_ = 1 - 0.0001

NUM_WORKERS = 32
ROWS_PER_CHUNK = 16
GROUP = 128
D = 16
LANE = 128


def _project_chunk(gbuf, sbuf, m, seq_len):
    lane = lax.iota(jnp.int32, 16)
    lanes = lane + m * ROWS_PER_CHUNK

    def block(j, carry):
        jj = jnp.full((16,), j, jnp.int32)
        diags = []
        ssum = jnp.zeros((16,), jnp.float32)
        for c in range(D):
            d2 = (lane + c) & (D - 1)
            dg = plsc.load_gather(gbuf, [lane, jj, d2])
            diags.append(dg)
            ssum = ssum + dg * dg
        bits = lax.bitcast_convert_type(ssum, jnp.int32)
        y = lax.bitcast_convert_type(
            jnp.int32(0x5F3759DF) - (bits >> 1), jnp.float32)
        for _ in range(3):
            y = y * (1.5 - 0.5 * ssum * y * y)
        norm = ssum * y
        factor = jnp.where(norm >= MAX_NORM_, MAX_NORM_ * y,
                           jnp.ones((16,), jnp.float32))
        for c in range(D):
            d2 = (lane + c) & (D - 1)
            plsc.store_scatter(sbuf, [jj * 2 + (d2 >> 3), d2 & 7, lanes],
                               diags[c] * factor)
        return carry

    lax.fori_loop(0, seq_len, block, 0)


def _make_sc_kernel(n_idx, seq_len):
    idx_rows_per_worker = n_idx // NUM_WORKERS
    groups = idx_rows_per_worker // GROUP
    chunks_per_group = GROUP // ROWS_PER_CHUNK
    tiles_per_group = 2 * seq_len
    info = plsc.get_sparse_core_info()
    nc = info.num_cores
    mesh = plsc.VectorSubcoreMesh(core_axis_name="c", subcore_axis_name="s")
    gbuf_t = pltpu.VMEM((ROWS_PER_CHUNK, seq_len, D), jnp.float32)
    sbuf_t = pltpu.VMEM((tiles_per_group, 8, LANE), jnp.float32)

    @functools.partial(
        pl.kernel,
        mesh=mesh,
        out_type=jax.ShapeDtypeStruct((tiles_per_group * (n_idx // GROUP),
                                       8, LANE), jnp.float32),
        compiler_params=pltpu.CompilerParams(needs_layout_passes=False,
                                             use_tc_tiling_on_sc=False),
        scratch_types=[
            pltpu.VMEM((seq_len, idx_rows_per_worker), jnp.int32),
            pltpu.VMEM((idx_rows_per_worker, seq_len), jnp.int32),
            gbuf_t,
            gbuf_t,
            sbuf_t,
            pltpu.SemaphoreType.DMA,
            pltpu.SemaphoreType.DMA,
            pltpu.SemaphoreType.DMA,
        ],
    )
    def sc_kernel(idx_hbm, emb_hbm, out_hbm, idx_tv, idx_v, ga, gb, sa,
                  gsem_a, gsem_b, ssem_a):
        wid = lax.axis_index("s") * nc + lax.axis_index("c")
        base = wid * idx_rows_per_worker
        pltpu.sync_copy(idx_hbm.at[:, pl.ds(base, idx_rows_per_worker)],
                        idx_tv)
        lane = lax.iota(jnp.int32, 16)

        def repack(c, carry):
            col = c * 16 + lane
            for j in range(seq_len):
                jj = jnp.full((16,), j, jnp.int32)
                v = plsc.load_gather(idx_tv, [jj, col])
                plsc.store_scatter(idx_v, [col, jj], v)
            return carry

        lax.fori_loop(0, idx_rows_per_worker // 16, repack, 0)

        def start_gathers(c, buf, gsem):
            r0 = c * ROWS_PER_CHUNK
            for k in range(ROWS_PER_CHUNK):
                pltpu.make_async_copy(
                    emb_hbm.at[idx_v.at[r0 + k]], buf.at[k], gsem).start()

        def wait_gathers(c, buf, gsem):
            r0 = c * ROWS_PER_CHUNK
            for k in range(ROWS_PER_CHUNK):
                pltpu.make_async_copy(
                    emb_hbm.at[idx_v.at[r0 + k]], buf.at[k], gsem).wait()

        def store_group(g, buf, ssem, wait):
            blk0 = (base // GROUP) + g
            for t in range(tiles_per_group):
                cp = pltpu.make_async_copy(
                    buf.at[t], out_hbm.at[t * (n_idx // GROUP) + blk0], ssem)
                if wait:
                    cp.wait()
                else:
                    cp.start()

        start_gathers(0, ga, gsem_a)

        def chunkstep(c, gbuf, gsem, ngbuf, ngsem):
            wait_gathers(c, gbuf, gsem)

            @pl.when(c + 1 < groups * chunks_per_group)
            def _():
                start_gathers(c + 1, ngbuf, ngsem)

            _project_chunk(gbuf, sa, c % chunks_per_group, seq_len)

        def gstep(g, carry):
            @pl.when(g >= 1)
            def _():
                store_group(g - 1, sa, ssem_a, wait=True)

            for h in range(chunks_per_group // 2):
                c = g * chunks_per_group + 2 * h
                chunkstep(c, ga, gsem_a, gb, gsem_b)
                chunkstep(c + 1, gb, gsem_b, ga, gsem_a)
            store_group(g, sa, ssem_a, wait=False)
            return carry

        lax.fori_loop(0, groups, gstep, 0)
        store_group(groups - 1, sa, ssem_a, wait=True)

    return sc_kernel


def kernel(idx, emb):
    n_idx, seq_len = idx.shape
    out4 = _make_sc_kernel(n_idx, seq_len)(idx.astype(jnp.int32).T, emb)
    out5 = out4.reshape(seq_len, 2, n_idx // GROUP, 8, LANE)
    return out5.transpose(2, 4, 0, 1, 3).reshape(n_idx, seq_len, D)
